# Initial kernel scaffold; baseline (speedup 1.0000x reference)
import functools

import jax
import jax.numpy as jnp
from jax import lax
from jax.experimental import pallas as pl
from jax.experimental.pallas import tpu as pltpu
from jax.experimental.pallas import tpu_sc as plsc

N = 100000
E = 1600000
B = 256
NT = 2000
DE = 16
DH = 64

SLAB = 25000
PAD = 8
EPT = E // 16
BLK = 10000
NBLK = EPT // BLK
CH = 128
CAP = BLK + 2 * CH

_MESH = dict(core_axis_name="c", subcore_axis_name="s")


def _bcast_lane(v16, l):
  return v16.at[jnp.full((16,), l, jnp.int32)].get(mode="promise_in_bounds")


def _msg_body(din, hin, srcr, dstr, ear, wer, ber, out,
              slab, dvm, svm, avm, csrc, cldst, cea, grows, idx2, wvm, sem):
  c = lax.axis_index("c")
  s = lax.axis_index("s")
  nj = din // 16
  pltpu.sync_copy(wer, wvm.at[pl.ds(0, din)])
  pltpu.sync_copy(ber, wvm.at[pl.ds(din, din)])
  wregs = [wvm[pl.ds(j * 16, 16)] for j in range(nj)]
  bregs = [wvm[pl.ds(din + j * 16, 16)] for j in range(nj)]
  iot = lax.iota(jnp.int32, 16)
  padd = SLAB + (iot & 7)
  ebase = s * EPT
  zero16 = jnp.zeros((16,), jnp.float32)

  for p in range(2):
    slab_id = c * 2 + p
    lo = slab_id * SLAB
    hi = lo + SLAB
    rbase = jnp.minimum(s * 1563, SLAB - 1568)
    pltpu.sync_copy(hin.at[pl.ds(lo + rbase, 1568)], slab.at[pl.ds(rbase, 1568)])
    plsc.subcore_barrier()

    def do_block(blk, _):
      eb = ebase + blk * BLK
      pltpu.sync_copy(dstr.at[pl.ds(eb, BLK)], dvm)
      pltpu.sync_copy(srcr.at[pl.ds(eb, BLK)], svm)
      pltpu.sync_copy(ear.at[pl.ds(eb, BLK)], avm)

      def compress(i, cnt):
        d = dvm[pl.ds(i * 16, 16)]
        m = (d >= lo) & (d < hi)
        plsc.store_compressed(cldst.at[pl.ds(cnt, 16)], d - lo, m)
        plsc.store_compressed(csrc.at[pl.ds(cnt, 16)], svm[pl.ds(i * 16, 16)], m)
        plsc.store_compressed(cea.at[pl.ds(cnt, 16)], avm[pl.ds(i * 16, 16)], m)
        return cnt + jnp.sum(m.astype(jnp.int32))

      cnt = lax.fori_loop(0, BLK // 16, compress, jnp.int32(0))

      def padw(i, carry):
        cldst[pl.ds(cnt + i * 16, 16)] = padd
        csrc[pl.ds(cnt + i * 16, 16)] = iot
        cea[pl.ds(cnt + i * 16, 16)] = zero16
        return carry
      lax.fori_loop(0, CH // 16, padw, 0)

      nsub = (cnt + (CH - 1)) // CH

      def chunk(k, carry):
        kb = k * CH
        pltpu.async_copy(hin.at[csrc.at[pl.ds(kb, CH)]], grows, sem).wait()

        def cpyidx(i, cc):
          idx2.at[0][pl.ds(i * 16, 16)] = cldst[pl.ds(kb + i * 16, 16)]
          return cc
        lax.fori_loop(0, CH // 16, cpyidx, 0)

        def grp(g, cc):
          ea16 = cea[pl.ds(kb + g * 16, 16)]
          for l in range(16):
            eab = _bcast_lane(ea16, l)
            rref = grows.at[g * 16 + l]
            for j in range(nj):
              row = rref[pl.ds(j * 16, 16)]
              rref[pl.ds(j * 16, 16)] = jnp.maximum(
                  row + eab * wregs[j] + bregs[j], 0.0)
          return cc
        lax.fori_loop(0, CH // 16, grp, 0)

        pltpu.sync_copy(grows, slab.at[idx2.at[0]], add=True)
        return carry
      lax.fori_loop(0, nsub, chunk, 0)
      return _

    lax.fori_loop(0, NBLK, do_block, 0)
    plsc.subcore_barrier()
    wb = jnp.minimum(s * 1563, SLAB - 1568)
    pltpu.sync_copy(slab.at[pl.ds(wb, 1568)], out.at[pl.ds(lo + wb, 1568)])
    plsc.subcore_barrier()


def _gine_msg(hin, src, dst, ea, w, b, din):
  body = functools.partial(_msg_body, din)
  kfn = pl.kernel(
      body,
      out_type=jax.ShapeDtypeStruct((N, din), jnp.float32),
      mesh=plsc.VectorSubcoreMesh(**_MESH),
      scratch_types=[
          pltpu.VMEM_SHARED((SLAB + PAD, din), jnp.float32),
          pltpu.VMEM((BLK,), jnp.int32),
          pltpu.VMEM((BLK,), jnp.int32),
          pltpu.VMEM((BLK,), jnp.float32),
          pltpu.VMEM((CAP,), jnp.int32),
          pltpu.VMEM((CAP,), jnp.int32),
          pltpu.VMEM((CAP,), jnp.float32),
          pltpu.VMEM((CH, din), jnp.float32),
          pltpu.VMEM((1, CH), jnp.int32),
          pltpu.VMEM((2 * din,), jnp.float32),
          pltpu.SemaphoreType.DMA,
      ],
  )
  return kfn(hin, src, dst, ea, w, b)


def _emb_body(emb, xi, out, idxv, rows, idxt, rowst, sem):
  c = lax.axis_index("c")
  s = lax.axis_index("s")
  wid = s * 2 + c
  nfull = N // CH
  nk = (nfull - wid + 31) // 32

  def do_chunk(k, carry):
    base = (wid + k * 32) * CH
    pltpu.sync_copy(xi.at[pl.ds(base, CH)], idxv)
    pltpu.async_copy(emb.at[idxv], rows, sem).wait()
    pltpu.sync_copy(rows, out.at[pl.ds(base, CH)])
    return carry
  lax.fori_loop(0, nk, do_chunk, 0)

  @pl.when(wid == 0)
  def _tail():
    base = nfull * CH
    pltpu.sync_copy(xi.at[pl.ds(base, 32)], idxt)
    pltpu.async_copy(emb.at[idxt], rowst, sem).wait()
    pltpu.sync_copy(rowst, out.at[pl.ds(base, 32)])


def _emb_lookup(emb, xi):
  kfn = pl.kernel(
      _emb_body,
      out_type=jax.ShapeDtypeStruct((N, DE), jnp.float32),
      mesh=plsc.VectorSubcoreMesh(**_MESH),
      scratch_types=[
          pltpu.VMEM((CH,), jnp.int32),
          pltpu.VMEM((CH, DE), jnp.float32),
          pltpu.VMEM((32,), jnp.int32),
          pltpu.VMEM((32, DE), jnp.float32),
          pltpu.SemaphoreType.DMA,
      ],
  )
  return kfn(emb, xi)


def _pool_body(h1, h2, h3, bat, out, pslab, zbuf, rows, bvm, idx2,
               rowst, bvt, idxt):
  c = lax.axis_index("c")
  s = lax.axis_index("s")

  @pl.when(s == 0)
  def _init():
    def zb(i, carry):
      zbuf.at[i // 4][pl.ds((i % 4) * 16, 16)] = jnp.zeros((16,), jnp.float32)
      return carry
    lax.fori_loop(0, (B + PAD) * 4, zb, 0)
    for l in range(3):
      pltpu.sync_copy(zbuf, pslab.at[l])
  plsc.subcore_barrier()

  rows_per_sc = N // 2
  nfull = rows_per_sc // CH
  base0 = c * rows_per_sc

  def do_chunk(k, carry):
    base = base0 + (s + k * 16) * CH
    pltpu.sync_copy(bat.at[pl.ds(base, CH)], bvm)

    def cpyidx(i, cc):
      idx2.at[0][pl.ds(i * 16, 16)] = bvm[pl.ds(i * 16, 16)]
      return cc
    lax.fori_loop(0, CH // 16, cpyidx, 0)
    for l, h in enumerate((h1, h2, h3)):
      pltpu.sync_copy(h.at[pl.ds(base, CH)], rows)
      pltpu.sync_copy(rows, pslab.at[l].at[idx2.at[0]], add=True)
    return carry

  nk = (nfull - s + 15) // 16
  lax.fori_loop(0, nk, do_chunk, 0)

  @pl.when(s == 0)
  def _tail():
    base = base0 + nfull * CH
    pltpu.sync_copy(bat.at[pl.ds(base, 80)], bvt)

    def cpyidx(i, cc):
      idxt.at[0][pl.ds(i * 16, 16)] = bvt[pl.ds(i * 16, 16)]
      return cc
    lax.fori_loop(0, 5, cpyidx, 0)
    for l, h in enumerate((h1, h2, h3)):
      pltpu.sync_copy(h.at[pl.ds(base, 80)], rowst)
      pltpu.sync_copy(rowst, pslab.at[l].at[idxt.at[0]], add=True)

  plsc.subcore_barrier()

  @pl.when(s == 0)
  def _writeout():
    for l in range(3):
      pltpu.sync_copy(pslab.at[l].at[pl.ds(0, B)], out.at[c].at[l])


def _pool(h1, h2, h3, bat):
  kfn = pl.kernel(
      _pool_body,
      out_type=jax.ShapeDtypeStruct((2, 3, B, DH), jnp.float32),
      mesh=plsc.VectorSubcoreMesh(**_MESH),
      scratch_types=[
          pltpu.VMEM_SHARED((3, B + PAD, DH), jnp.float32),
          pltpu.VMEM((B + PAD, DH), jnp.float32),
          pltpu.VMEM((CH, DH), jnp.float32),
          pltpu.VMEM((CH,), jnp.int32),
          pltpu.VMEM((1, CH), jnp.int32),
          pltpu.VMEM((80, DH), jnp.float32),
          pltpu.VMEM((80,), jnp.int32),
          pltpu.VMEM((1, 80), jnp.int32),
      ],
  )
  return kfn(h1, h2, h3, bat)


_BR = 5000


def _stats_body(din, h_ref, o_ref):
  i = pl.program_id(0)

  @pl.when(i == 0)
  def _():
    o_ref[...] = jnp.zeros_like(o_ref)

  hb = h_ref[...]
  g = lax.dot_general(hb, hb, (((0,), (0,)), ((), ())),
                      preferred_element_type=jnp.float32)
  cs = jnp.sum(hb, axis=0, keepdims=True)
  o_ref[...] += jnp.concatenate(
      [g, cs, jnp.zeros((7, din), jnp.float32)], axis=0)


def _stats(h, din):
  return pl.pallas_call(
      functools.partial(_stats_body, din),
      grid=(N // _BR,),
      in_specs=[pl.BlockSpec((_BR, din), lambda i: (i, 0))],
      out_specs=pl.BlockSpec((din + 8, din), lambda i: (0, 0)),
      out_shape=jax.ShapeDtypeStruct((din + 8, din), jnp.float32),
  )(h)


def _mlp_body(din, h_ref, gs_ref, w1_ref, g_ref, bt_ref, w2_ref, b2_ref, o_ref):
  G = gs_ref[0:din, :]
  m = gs_ref[din:din + 1, :] * (1.0 / N)
  W1 = w1_ref[...]
  GW = jnp.dot(G, W1, preferred_element_type=jnp.float32)
  q = jnp.sum(W1 * GW, axis=0, keepdims=True) * (1.0 / N)
  mz0 = jnp.dot(m, W1, preferred_element_type=jnp.float32)
  varz = q - mz0 * mz0
  sc = g_ref[...] * lax.rsqrt(varz + 1e-5)
  W1e = W1 * sc
  b1e = bt_ref[...] - mz0 * sc
  hb = h_ref[...]
  t = jnp.maximum(jnp.dot(hb, W1e, preferred_element_type=jnp.float32) + b1e, 0.0)
  o_ref[...] = jnp.maximum(
      jnp.dot(t, w2_ref[...], preferred_element_type=jnp.float32) + b2_ref[...],
      0.0)


def _mlp(h, gs, W1, g, bt, W2, b2, din):
  full = lambda r, c: pl.BlockSpec((r, c), lambda i: (0, 0))
  return pl.pallas_call(
      functools.partial(_mlp_body, din),
      grid=(N // _BR,),
      in_specs=[
          pl.BlockSpec((_BR, din), lambda i: (i, 0)),
          full(din + 8, din),
          full(din, DH),
          full(1, DH),
          full(1, DH),
          full(DH, DH),
          full(1, DH),
      ],
      out_specs=pl.BlockSpec((_BR, DH), lambda i: (i, 0)),
      out_shape=jax.ShapeDtypeStruct((N, DH), jnp.float32),
  )(h, gs, W1, g.reshape(1, DH), bt.reshape(1, DH), W2, b2.reshape(1, DH))


def _heads_body(p_ref, fw1, fb1, fw2, fb2, bw1, bb1, bw2, bb2, of_ref, ob_ref):
  p = p_ref[...]
  ps = p[0] + p[1]
  h = jnp.concatenate([ps[0], ps[1], ps[2]], axis=1)
  tf = jnp.maximum(jnp.dot(h, fw1[...], preferred_element_type=jnp.float32)
                   + fb1[...], 0.0)
  of_ref[...] = jnp.dot(tf, fw2[...], preferred_element_type=jnp.float32) + fb2[...]
  tb = jnp.maximum(jnp.dot(h, bw1[...], preferred_element_type=jnp.float32)
                   + bb1[...], 0.0)
  ob_ref[...] = jnp.dot(tb, bw2[...], preferred_element_type=jnp.float32) + bb2[...]


def _heads(pooled, f_W1, f_b1, f_W2, f_b2, b_W1, b_b1, b_W2, b_b2):
  def full(*sh):
    return pl.BlockSpec(sh, lambda: tuple(0 for _ in sh))
  return pl.pallas_call(
      _heads_body,
      in_specs=[
          full(2, 3, B, DH),
          full(3 * DH, DH), full(1, DH), full(DH, NT), full(1, NT),
          full(3 * DH, DH), full(1, DH), full(DH, NT), full(1, NT),
      ],
      out_specs=[full(B, NT), full(B, NT)],
      out_shape=[jax.ShapeDtypeStruct((B, NT), jnp.float32),
                 jax.ShapeDtypeStruct((B, NT), jnp.float32)],
  )(pooled, f_W1, f_b1.reshape(1, NT), f_W2, f_b2.reshape(1, NT),
    b_W1, b_b1.reshape(1, NT), b_W2, b_b2.reshape(1, NT))


def kernel(x, edge_index, edge_attr, batch, emb,
           c1_We, c1_be, c1_W1, c1_b1, c1_g, c1_bt, c1_W2, c1_b2,
           c2_We, c2_be, c2_W1, c2_b1, c2_g, c2_bt, c2_W2, c2_b2,
           c3_We, c3_be, c3_W1, c3_b1, c3_g, c3_bt, c3_W2, c3_b2,
           f_W1, f_b1, f_W2, f_b2, b_W1, b_b1, b_W2, b_b2):
  xi = x[:, 0].astype(jnp.int32)
  src = edge_index[0].astype(jnp.int32)
  dst = edge_index[1].astype(jnp.int32)
  ea = edge_attr[:, 0].astype(jnp.float32)

  h0 = _emb_lookup(emb, xi)
  hs1 = _gine_msg(h0, src, dst, ea, c1_We[0], c1_be, DE)
  h1 = _mlp(hs1, _stats(hs1, DE), c1_W1, c1_g, c1_bt, c1_W2, c1_b2, DE)
  hs2 = _gine_msg(h1, src, dst, ea, c2_We[0], c2_be, DH)
  h2 = _mlp(hs2, _stats(hs2, DH), c2_W1, c2_g, c2_bt, c2_W2, c2_b2, DH)
  hs3 = _gine_msg(h2, src, dst, ea, c3_We[0], c3_be, DH)
  h3 = _mlp(hs3, _stats(hs3, DH), c3_W1, c3_g, c3_bt, c3_W2, c3_b2, DH)
  pooled = _pool(h1, h2, h3, batch.astype(jnp.int32))
  return _heads(pooled, f_W1, f_b1, f_W2, f_b2, b_W1, b_b1, b_W2, b_b2)


# speedup vs baseline: 5.5552x; 5.5552x over previous
import functools

import jax
import jax.numpy as jnp
from jax import lax
from jax.experimental import pallas as pl
from jax.experimental.pallas import tpu as pltpu
from jax.experimental.pallas import tpu_sc as plsc

N = 100000
E = 1600000
B = 256
NT = 2000
DE = 16
DH = 64

SLAB = 25000
PAD = 128
EPT = E // 16
BLK = 2000
NBLK = EPT // BLK
CH = 128
CAP = BLK + 2 * CH

_MESH = dict(core_axis_name="c", subcore_axis_name="s")


def _bcast_lane(v16, l):
  return v16.at[jnp.full((16,), l, jnp.int32)].get(mode="promise_in_bounds")


def _msg_body(din, hin, srcr, dstr, ear, wer, ber, out,
              slab, dvm, svm, avm, csrc, cldst, cea, grows, idx2, wvm, sem):
  c = lax.axis_index("c")
  s = lax.axis_index("s")
  nj = din // 16
  pltpu.sync_copy(wer, wvm.at[pl.ds(0, din)])
  pltpu.sync_copy(ber, wvm.at[pl.ds(din, din)])
  wregs = [wvm[pl.ds(j * 16, 16)] for j in range(nj)]
  bregs = [wvm[pl.ds(din + j * 16, 16)] for j in range(nj)]
  iot = lax.iota(jnp.int32, 16)
  padd = SLAB + s * 8 + (iot & 7)
  psrc = s * 16 + iot
  ebase = s * EPT
  zero16 = jnp.zeros((16,), jnp.float32)

  for p in range(2):
    slab_id = c * 2 + p
    lo = slab_id * SLAB
    hi = lo + SLAB
    rbase = s * 1562
    pltpu.sync_copy(hin.at[pl.ds(lo + rbase, 1562)], slab.at[pl.ds(rbase, 1562)])

    @pl.when(s == 0)
    def _init_tail():
      pltpu.sync_copy(hin.at[pl.ds(lo + 24992, 8)], slab.at[pl.ds(24992, 8)])
    plsc.subcore_barrier()

    def do_block(blk, _):
      eb = ebase + blk * BLK
      pltpu.sync_copy(dstr.at[pl.ds(eb, BLK)], dvm)
      pltpu.sync_copy(srcr.at[pl.ds(eb, BLK)], svm)
      pltpu.sync_copy(ear.at[pl.ds(eb, BLK)], avm)

      def compress(i, cnt):
        d = dvm[pl.ds(i * 16, 16)]
        m = (d >= lo) & (d < hi)
        plsc.store_compressed(cldst.at[pl.ds(cnt, 16)], d - lo, mask=m)
        plsc.store_compressed(csrc.at[pl.ds(cnt, 16)], svm[pl.ds(i * 16, 16)], mask=m)
        plsc.store_compressed(cea.at[pl.ds(cnt, 16)], avm[pl.ds(i * 16, 16)], mask=m)
        return cnt + jnp.sum(m.astype(jnp.int32))

      cnt = lax.fori_loop(0, BLK // 16, compress, jnp.int32(0))

      tru = iot == iot

      def padw(i, carry):
        off = pl.ds(cnt + i * 16, 16)
        plsc.store_compressed(cldst.at[off], padd, mask=tru)
        plsc.store_compressed(csrc.at[off], psrc, mask=tru)
        plsc.store_compressed(cea.at[off], zero16, mask=tru)
        return carry
      lax.fori_loop(0, CH // 16, padw, 0)

      nsub = (cnt + (CH - 1)) // CH

      def chunk(k, carry):
        kb = k * CH
        pltpu.async_copy(hin.at[csrc.at[pl.ds(kb, CH)]], grows, sem).wait()

        def cpyidx(i, cc):
          idx2.at[0][pl.ds(i * 16, 16)] = cldst[pl.ds(kb + i * 16, 16)]
          return cc
        lax.fori_loop(0, CH // 16, cpyidx, 0)

        def grp(g, cc):
          ea16 = cea[pl.ds(kb + g * 16, 16)]
          for l in range(16):
            eab = _bcast_lane(ea16, l)
            rref = grows.at[g * 16 + l]
            for j in range(nj):
              row = rref[pl.ds(j * 16, 16)]
              rref[pl.ds(j * 16, 16)] = jnp.maximum(
                  row + eab * wregs[j] + bregs[j], 0.0)
          return cc
        lax.fori_loop(0, CH // 16, grp, 0)

        pltpu.sync_copy(grows, slab.at[idx2.at[0]], add=True)
        return carry
      lax.fori_loop(0, nsub, chunk, 0)
      return _

    lax.fori_loop(0, NBLK, do_block, 0)
    plsc.subcore_barrier()
    wb = s * 1562
    pltpu.sync_copy(slab.at[pl.ds(wb, 1562)], out.at[pl.ds(lo + wb, 1562)])

    @pl.when(s == 0)
    def _wout_tail():
      pltpu.sync_copy(slab.at[pl.ds(24992, 8)], out.at[pl.ds(lo + 24992, 8)])
    plsc.subcore_barrier()


def _gine_msg(hin, src, dst, ea, w, b, din):
  body = functools.partial(_msg_body, din)
  kfn = pl.kernel(
      body,
      out_type=jax.ShapeDtypeStruct((N, din), jnp.float32),
      mesh=plsc.VectorSubcoreMesh(**_MESH),
      compiler_params=pltpu.CompilerParams(use_tc_tiling_on_sc=False, needs_layout_passes=False),
      scratch_types=[
          pltpu.VMEM_SHARED((SLAB + PAD, din), jnp.float32),
          pltpu.VMEM((BLK,), jnp.int32),
          pltpu.VMEM((BLK,), jnp.int32),
          pltpu.VMEM((BLK,), jnp.float32),
          pltpu.VMEM((CAP,), jnp.int32),
          pltpu.VMEM((CAP,), jnp.int32),
          pltpu.VMEM((CAP,), jnp.float32),
          pltpu.VMEM((CH, din), jnp.float32),
          pltpu.VMEM((1, CH), jnp.int32),
          pltpu.VMEM((2 * din,), jnp.float32),
          pltpu.SemaphoreType.DMA,
      ],
  )
  return kfn(hin, src, dst, ea, w, b)


def _emb_body(emb, xi, out, idxv, rows, idxt, rowst, sem):
  c = lax.axis_index("c")
  s = lax.axis_index("s")
  wid = s * 2 + c
  nfull = N // CH
  nk = (nfull - wid + 31) // 32

  def do_chunk(k, carry):
    base = (wid + k * 32) * CH
    pltpu.sync_copy(xi.at[pl.ds(base, CH)], idxv)
    pltpu.async_copy(emb.at[idxv], rows, sem).wait()
    pltpu.sync_copy(rows, out.at[pl.ds(base, CH)])
    return carry
  lax.fori_loop(0, nk, do_chunk, 0)

  @pl.when(wid == 0)
  def _tail():
    base = nfull * CH
    pltpu.sync_copy(xi.at[pl.ds(base, 32)], idxt)
    pltpu.async_copy(emb.at[idxt], rowst, sem).wait()
    pltpu.sync_copy(rowst, out.at[pl.ds(base, 32)])


def _emb_lookup(emb, xi):
  kfn = pl.kernel(
      _emb_body,
      out_type=jax.ShapeDtypeStruct((N, DE), jnp.float32),
      mesh=plsc.VectorSubcoreMesh(**_MESH),
      compiler_params=pltpu.CompilerParams(use_tc_tiling_on_sc=False, needs_layout_passes=False),
      scratch_types=[
          pltpu.VMEM((CH,), jnp.int32),
          pltpu.VMEM((CH, DE), jnp.float32),
          pltpu.VMEM((32,), jnp.int32),
          pltpu.VMEM((32, DE), jnp.float32),
          pltpu.SemaphoreType.DMA,
      ],
  )
  return kfn(emb, xi)


def _pool_body(h1, h2, h3, bat, out, pslab, zbuf, rows, bvm, idx2,
               rowst, bvt, idxt):
  c = lax.axis_index("c")
  s = lax.axis_index("s")

  @pl.when(s == 0)
  def _init():
    def zb(i, carry):
      zbuf.at[i // 4][pl.ds((i % 4) * 16, 16)] = jnp.zeros((16,), jnp.float32)
      return carry
    lax.fori_loop(0, (B + PAD) * 4, zb, 0)
    for l in range(3):
      pltpu.sync_copy(zbuf, pslab.at[l])
  plsc.subcore_barrier()

  rows_per_sc = N // 2
  nfull = rows_per_sc // CH
  base0 = c * rows_per_sc

  def do_chunk(k, carry):
    base = base0 + (s + k * 16) * CH
    pltpu.sync_copy(bat.at[pl.ds(base, CH)], bvm)

    def cpyidx(i, cc):
      idx2.at[0][pl.ds(i * 16, 16)] = bvm[pl.ds(i * 16, 16)]
      return cc
    lax.fori_loop(0, CH // 16, cpyidx, 0)
    for l, h in enumerate((h1, h2, h3)):
      pltpu.sync_copy(h.at[pl.ds(base, CH)], rows)
      pltpu.sync_copy(rows, pslab.at[l].at[idx2.at[0]], add=True)
    return carry

  nk = (nfull - s + 15) // 16
  lax.fori_loop(0, nk, do_chunk, 0)

  @pl.when(s == 0)
  def _tail():
    base = base0 + nfull * CH
    pltpu.sync_copy(bat.at[pl.ds(base, 80)], bvt)

    def cpyidx(i, cc):
      idxt.at[0][pl.ds(i * 16, 16)] = bvt[pl.ds(i * 16, 16)]
      return cc
    lax.fori_loop(0, 5, cpyidx, 0)
    for l, h in enumerate((h1, h2, h3)):
      pltpu.sync_copy(h.at[pl.ds(base, 80)], rowst)
      pltpu.sync_copy(rowst, pslab.at[l].at[idxt.at[0]], add=True)

  plsc.subcore_barrier()

  @pl.when(s == 0)
  def _writeout():
    for l in range(3):
      pltpu.sync_copy(pslab.at[l].at[pl.ds(0, B)], out.at[c].at[l])


def _pool(h1, h2, h3, bat):
  kfn = pl.kernel(
      _pool_body,
      out_type=jax.ShapeDtypeStruct((2, 3, B, DH), jnp.float32),
      mesh=plsc.VectorSubcoreMesh(**_MESH),
      compiler_params=pltpu.CompilerParams(use_tc_tiling_on_sc=False, needs_layout_passes=False),
      scratch_types=[
          pltpu.VMEM_SHARED((3, B + PAD, DH), jnp.float32),
          pltpu.VMEM((B + PAD, DH), jnp.float32),
          pltpu.VMEM((CH, DH), jnp.float32),
          pltpu.VMEM((CH,), jnp.int32),
          pltpu.VMEM((1, CH), jnp.int32),
          pltpu.VMEM((80, DH), jnp.float32),
          pltpu.VMEM((80,), jnp.int32),
          pltpu.VMEM((1, 80), jnp.int32),
      ],
  )
  return kfn(h1, h2, h3, bat)


_BR = 5000


def _z_body(din, h_ref, w1_ref, b1_ref, z_ref, s_ref):
  i = pl.program_id(0)

  @pl.when(i == 0)
  def _():
    s_ref[...] = jnp.zeros_like(s_ref)

  z = jnp.dot(h_ref[...], w1_ref[...],
              preferred_element_type=jnp.float32) + b1_ref[...]
  z_ref[...] = z
  s_ref[...] += jnp.concatenate(
      [jnp.sum(z, axis=0, keepdims=True), jnp.zeros((7, DH), jnp.float32)],
      axis=0)


def _zpass(h, W1, b1, din):
  full = lambda r, c: pl.BlockSpec((r, c), lambda i: (0, 0))
  return pl.pallas_call(
      functools.partial(_z_body, din),
      grid=(N // _BR,),
      in_specs=[
          pl.BlockSpec((_BR, din), lambda i: (i, 0)),
          full(din, DH),
          full(1, DH),
      ],
      out_specs=[pl.BlockSpec((_BR, DH), lambda i: (i, 0)), full(8, DH)],
      out_shape=[jax.ShapeDtypeStruct((N, DH), jnp.float32),
                 jax.ShapeDtypeStruct((8, DH), jnp.float32)],
  )(h, W1, b1.reshape(1, DH))


def _var_body(z_ref, s_ref, v_ref):
  i = pl.program_id(0)

  @pl.when(i == 0)
  def _():
    v_ref[...] = jnp.zeros_like(v_ref)

  m = s_ref[0:1, :] * (1.0 / N)
  zc = z_ref[...] - m
  v_ref[...] += jnp.concatenate(
      [jnp.sum(zc * zc, axis=0, keepdims=True),
       jnp.zeros((7, DH), jnp.float32)], axis=0)


def _varpass(z, s):
  full = lambda r, c: pl.BlockSpec((r, c), lambda i: (0, 0))
  return pl.pallas_call(
      _var_body,
      grid=(N // _BR,),
      in_specs=[pl.BlockSpec((_BR, DH), lambda i: (i, 0)), full(8, DH)],
      out_specs=full(8, DH),
      out_shape=jax.ShapeDtypeStruct((8, DH), jnp.float32),
  )(z, s)


def _bnmlp_body(z_ref, s_ref, v_ref, g_ref, bt_ref, w2_ref, b2_ref, o_ref):
  m = s_ref[0:1, :] * (1.0 / N)
  v = v_ref[0:1, :] * (1.0 / N)
  t = jnp.maximum(
      g_ref[...] * (z_ref[...] - m) / jnp.sqrt(v + 1e-5) + bt_ref[...], 0.0)
  o_ref[...] = jnp.maximum(
      jnp.dot(t, w2_ref[...], preferred_element_type=jnp.float32) + b2_ref[...],
      0.0)


def _bnmlp(z, s, v, g, bt, W2, b2):
  full = lambda r, c: pl.BlockSpec((r, c), lambda i: (0, 0))
  return pl.pallas_call(
      _bnmlp_body,
      grid=(N // _BR,),
      in_specs=[
          pl.BlockSpec((_BR, DH), lambda i: (i, 0)),
          full(8, DH), full(8, DH), full(1, DH), full(1, DH),
          full(DH, DH), full(1, DH),
      ],
      out_specs=pl.BlockSpec((_BR, DH), lambda i: (i, 0)),
      out_shape=jax.ShapeDtypeStruct((N, DH), jnp.float32),
  )(z, s, v, g.reshape(1, DH), bt.reshape(1, DH), W2, b2.reshape(1, DH))


def _mlp(h, W1, b1, g, bt, W2, b2, din):
  z, szsum = _zpass(h, W1, b1, din)
  vsum = _varpass(z, szsum)
  return _bnmlp(z, szsum, vsum, g, bt, W2, b2)


def _heads_body(p_ref, fw1, fb1, fw2, fb2, bw1, bb1, bw2, bb2, of_ref, ob_ref):
  p = p_ref[...]
  ps = p[0] + p[1]
  h = jnp.concatenate([ps[0], ps[1], ps[2]], axis=1)
  tf = jnp.maximum(jnp.dot(h, fw1[...], preferred_element_type=jnp.float32)
                   + fb1[...], 0.0)
  of_ref[...] = jnp.dot(tf, fw2[...], preferred_element_type=jnp.float32) + fb2[...]
  tb = jnp.maximum(jnp.dot(h, bw1[...], preferred_element_type=jnp.float32)
                   + bb1[...], 0.0)
  ob_ref[...] = jnp.dot(tb, bw2[...], preferred_element_type=jnp.float32) + bb2[...]


def _heads(pooled, f_W1, f_b1, f_W2, f_b2, b_W1, b_b1, b_W2, b_b2):
  def full(*sh):
    return pl.BlockSpec(sh, lambda: tuple(0 for _ in sh))
  return pl.pallas_call(
      _heads_body,
      in_specs=[
          full(2, 3, B, DH),
          full(3 * DH, DH), full(1, DH), full(DH, NT), full(1, NT),
          full(3 * DH, DH), full(1, DH), full(DH, NT), full(1, NT),
      ],
      out_specs=[full(B, NT), full(B, NT)],
      out_shape=[jax.ShapeDtypeStruct((B, NT), jnp.float32),
                 jax.ShapeDtypeStruct((B, NT), jnp.float32)],
  )(pooled, f_W1, f_b1.reshape(1, DH), f_W2, f_b2.reshape(1, NT),
    b_W1, b_b1.reshape(1, DH), b_W2, b_b2.reshape(1, NT))


def kernel(x, edge_index, edge_attr, batch, emb,
           c1_We, c1_be, c1_W1, c1_b1, c1_g, c1_bt, c1_W2, c1_b2,
           c2_We, c2_be, c2_W1, c2_b1, c2_g, c2_bt, c2_W2, c2_b2,
           c3_We, c3_be, c3_W1, c3_b1, c3_g, c3_bt, c3_W2, c3_b2,
           f_W1, f_b1, f_W2, f_b2, b_W1, b_b1, b_W2, b_b2):
  xi = x[:, 0].astype(jnp.int32)
  src = edge_index[0].astype(jnp.int32)
  dst = edge_index[1].astype(jnp.int32)
  ea = edge_attr[:, 0].astype(jnp.float32)

  h0 = _emb_lookup(emb, xi)
  hs1 = _gine_msg(h0, src, dst, ea, c1_We[0], c1_be, DE)
  h1 = _mlp(hs1, c1_W1, c1_b1, c1_g, c1_bt, c1_W2, c1_b2, DE)
  hs2 = _gine_msg(h1, src, dst, ea, c2_We[0], c2_be, DH)
  h2 = _mlp(hs2, c2_W1, c2_b1, c2_g, c2_bt, c2_W2, c2_b2, DH)
  hs3 = _gine_msg(h2, src, dst, ea, c3_We[0], c3_be, DH)
  h3 = _mlp(hs3, c3_W1, c3_b1, c3_g, c3_bt, c3_W2, c3_b2, DH)
  pooled = _pool(h1, h2, h3, batch.astype(jnp.int32))
  return _heads(pooled, f_W1, f_b1, f_W2, f_b2, b_W1, b_b1, b_W2, b_b2)


# speedup vs baseline: 6.6236x; 1.1923x over previous
import functools

import jax
import jax.numpy as jnp
from jax import lax
from jax.experimental import pallas as pl
from jax.experimental.pallas import tpu as pltpu
from jax.experimental.pallas import tpu_sc as plsc

N = 100000
E = 1600000
B = 256
NT = 2000
DE = 16
DH = 64

SLAB = 25000
PAD = 128
EPT = E // 16
BLK = 2000
NBLK = EPT // BLK
CH = 128
CAP = BLK + CH + 16

_MESH = dict(core_axis_name="c", subcore_axis_name="s")


def _bcast_lane(v16, l):
  return v16.at[jnp.full((16,), l, jnp.int32)].get(mode="promise_in_bounds")


def _msg_body(din, hin, srcr, dstr, ear, wer, ber, out,
              slab, dvm, svm, avm, csrc, cldst, cea, grows, grows2, idx2, wvm,
              sem, sem2):
  c = lax.axis_index("c")
  s = lax.axis_index("s")
  nj = din // 16
  pltpu.sync_copy(wer, wvm.at[pl.ds(0, din)])
  pltpu.sync_copy(ber, wvm.at[pl.ds(din, din)])
  wregs = [wvm[pl.ds(j * 16, 16)] for j in range(nj)]
  bregs = [wvm[pl.ds(din + j * 16, 16)] for j in range(nj)]
  iot = lax.iota(jnp.int32, 16)
  padd = SLAB + s * 8 + (iot & 7)
  psrc = s * 16 + iot
  ebase = s * EPT
  zero16 = jnp.zeros((16,), jnp.float32)

  for p in range(2):
    slab_id = c * 2 + p
    lo = slab_id * SLAB
    hi = lo + SLAB
    rbase = s * 1562
    pltpu.sync_copy(hin.at[pl.ds(lo + rbase, 1562)], slab.at[pl.ds(rbase, 1562)])

    @pl.when(s == 0)
    def _init_tail():
      pltpu.sync_copy(hin.at[pl.ds(lo + 24992, 8)], slab.at[pl.ds(24992, 8)])
    plsc.subcore_barrier()

    def do_block(blk, _):
      eb = ebase + blk * BLK
      pltpu.sync_copy(dstr.at[pl.ds(eb, BLK)], dvm)
      pltpu.sync_copy(srcr.at[pl.ds(eb, BLK)], svm)
      pltpu.sync_copy(ear.at[pl.ds(eb, BLK)], avm)

      def compress(i, cnt):
        d = dvm[pl.ds(i * 16, 16)]
        m = (d >= lo) & (d < hi)
        plsc.store_compressed(cldst.at[pl.ds(cnt, 16)], d - lo, mask=m)
        plsc.store_compressed(csrc.at[pl.ds(cnt, 16)], svm[pl.ds(i * 16, 16)], mask=m)
        plsc.store_compressed(cea.at[pl.ds(cnt, 16)], avm[pl.ds(i * 16, 16)], mask=m)
        return cnt + jnp.sum(m.astype(jnp.int32))

      cnt = lax.fori_loop(0, BLK // 16, compress, jnp.int32(0))

      tru = iot == iot

      def padw(i, carry):
        off = pl.ds(cnt + i * 16, 16)
        plsc.store_compressed(cldst.at[off], padd, mask=tru)
        plsc.store_compressed(csrc.at[off], psrc, mask=tru)
        plsc.store_compressed(cea.at[off], zero16, mask=tru)
        return carry
      lax.fori_loop(0, CH // 16, padw, 0)

      nsub = (cnt + (CH - 1)) // CH

      def fire(k, buf, sm):
        pltpu.async_copy(hin.at[csrc.at[pl.ds(k * CH, CH)]], buf, sm)

      def process(k, buf, sm, obuf, osm):
        pltpu.make_async_copy(hin.at[csrc.at[pl.ds(0, CH)]], buf, sm).wait()

        @pl.when(k + 1 < nsub)
        def _pref():
          fire(k + 1, obuf, osm)
        kb = k * CH

        def cpyidx(i, cc):
          idx2.at[0][pl.ds(i * 16, 16)] = cldst[pl.ds(kb + i * 16, 16)]
          return cc
        lax.fori_loop(0, CH // 16, cpyidx, 0)

        def grp(g, cc):
          ea16 = cea[pl.ds(kb + g * 16, 16)]
          for l in range(16):
            eab = _bcast_lane(ea16, l)
            rref = buf.at[g * 16 + l]
            for j in range(nj):
              row = rref[pl.ds(j * 16, 16)]
              rref[pl.ds(j * 16, 16)] = jnp.maximum(
                  row + eab * wregs[j] + bregs[j], 0.0)
          return cc
        lax.fori_loop(0, CH // 16, grp, 0)

        pltpu.sync_copy(buf, slab.at[idx2.at[0]], add=True)

      @pl.when(nsub > 0)
      def _prime():
        fire(0, grows, sem)

      def chunk(k, carry):
        @pl.when(k % 2 == 0)
        def _even():
          process(k, grows, sem, grows2, sem2)

        @pl.when(k % 2 == 1)
        def _odd():
          process(k, grows2, sem2, grows, sem)
        return carry
      lax.fori_loop(0, nsub, chunk, 0)
      return _

    lax.fori_loop(0, NBLK, do_block, 0)
    plsc.subcore_barrier()
    wb = s * 1562
    pltpu.sync_copy(slab.at[pl.ds(wb, 1562)], out.at[pl.ds(lo + wb, 1562)])

    @pl.when(s == 0)
    def _wout_tail():
      pltpu.sync_copy(slab.at[pl.ds(24992, 8)], out.at[pl.ds(lo + 24992, 8)])
    plsc.subcore_barrier()


def _gine_msg(hin, src, dst, ea, w, b, din):
  body = functools.partial(_msg_body, din)
  kfn = pl.kernel(
      body,
      out_type=jax.ShapeDtypeStruct((N, din), jnp.float32),
      mesh=plsc.VectorSubcoreMesh(**_MESH),
      compiler_params=pltpu.CompilerParams(use_tc_tiling_on_sc=False, needs_layout_passes=False),
      scratch_types=[
          pltpu.VMEM_SHARED((SLAB + PAD, din), jnp.float32),
          pltpu.VMEM((BLK,), jnp.int32),
          pltpu.VMEM((BLK,), jnp.int32),
          pltpu.VMEM((BLK,), jnp.float32),
          pltpu.VMEM((CAP,), jnp.int32),
          pltpu.VMEM((CAP,), jnp.int32),
          pltpu.VMEM((CAP,), jnp.float32),
          pltpu.VMEM((CH, din), jnp.float32),
          pltpu.VMEM((CH, din), jnp.float32),
          pltpu.VMEM((1, CH), jnp.int32),
          pltpu.VMEM((2 * din,), jnp.float32),
          pltpu.SemaphoreType.DMA,
          pltpu.SemaphoreType.DMA,
      ],
  )
  return kfn(hin, src, dst, ea, w, b)


def _emb_body(emb, xi, out, idxv, rows, idxt, rowst, sem):
  c = lax.axis_index("c")
  s = lax.axis_index("s")
  wid = s * 2 + c
  nfull = N // CH
  nk = (nfull - wid + 31) // 32

  def do_chunk(k, carry):
    base = (wid + k * 32) * CH
    pltpu.sync_copy(xi.at[pl.ds(base, CH)], idxv)
    pltpu.async_copy(emb.at[idxv], rows, sem).wait()
    pltpu.sync_copy(rows, out.at[pl.ds(base, CH)])
    return carry
  lax.fori_loop(0, nk, do_chunk, 0)

  @pl.when(wid == 0)
  def _tail():
    base = nfull * CH
    pltpu.sync_copy(xi.at[pl.ds(base, 32)], idxt)
    pltpu.async_copy(emb.at[idxt], rowst, sem).wait()
    pltpu.sync_copy(rowst, out.at[pl.ds(base, 32)])


def _emb_lookup(emb, xi):
  kfn = pl.kernel(
      _emb_body,
      out_type=jax.ShapeDtypeStruct((N, DE), jnp.float32),
      mesh=plsc.VectorSubcoreMesh(**_MESH),
      compiler_params=pltpu.CompilerParams(use_tc_tiling_on_sc=False, needs_layout_passes=False),
      scratch_types=[
          pltpu.VMEM((CH,), jnp.int32),
          pltpu.VMEM((CH, DE), jnp.float32),
          pltpu.VMEM((32,), jnp.int32),
          pltpu.VMEM((32, DE), jnp.float32),
          pltpu.SemaphoreType.DMA,
      ],
  )
  return kfn(emb, xi)


def _pool_body(h1, h2, h3, bat, out, pslab, zbuf, rows, bvm, idx2,
               rowst, bvt, idxt):
  c = lax.axis_index("c")
  s = lax.axis_index("s")

  @pl.when(s == 0)
  def _init():
    def zb(i, carry):
      zbuf.at[i // 4][pl.ds((i % 4) * 16, 16)] = jnp.zeros((16,), jnp.float32)
      return carry
    lax.fori_loop(0, (B + PAD) * 4, zb, 0)
    for l in range(3):
      pltpu.sync_copy(zbuf, pslab.at[l])
  plsc.subcore_barrier()

  rows_per_sc = N // 2
  nfull = rows_per_sc // CH
  base0 = c * rows_per_sc

  def do_chunk(k, carry):
    base = base0 + (s + k * 16) * CH
    pltpu.sync_copy(bat.at[pl.ds(base, CH)], bvm)

    def cpyidx(i, cc):
      idx2.at[0][pl.ds(i * 16, 16)] = bvm[pl.ds(i * 16, 16)]
      return cc
    lax.fori_loop(0, CH // 16, cpyidx, 0)
    for l, h in enumerate((h1, h2, h3)):
      pltpu.sync_copy(h.at[pl.ds(base, CH)], rows)
      pltpu.sync_copy(rows, pslab.at[l].at[idx2.at[0]], add=True)
    return carry

  nk = (nfull - s + 15) // 16
  lax.fori_loop(0, nk, do_chunk, 0)

  @pl.when(s == 0)
  def _tail():
    base = base0 + nfull * CH
    pltpu.sync_copy(bat.at[pl.ds(base, 80)], bvt)

    def cpyidx(i, cc):
      idxt.at[0][pl.ds(i * 16, 16)] = bvt[pl.ds(i * 16, 16)]
      return cc
    lax.fori_loop(0, 5, cpyidx, 0)
    for l, h in enumerate((h1, h2, h3)):
      pltpu.sync_copy(h.at[pl.ds(base, 80)], rowst)
      pltpu.sync_copy(rowst, pslab.at[l].at[idxt.at[0]], add=True)

  plsc.subcore_barrier()

  @pl.when(s == 0)
  def _writeout():
    for l in range(3):
      pltpu.sync_copy(pslab.at[l].at[pl.ds(0, B)], out.at[c].at[l])


def _pool(h1, h2, h3, bat):
  kfn = pl.kernel(
      _pool_body,
      out_type=jax.ShapeDtypeStruct((2, 3, B, DH), jnp.float32),
      mesh=plsc.VectorSubcoreMesh(**_MESH),
      compiler_params=pltpu.CompilerParams(use_tc_tiling_on_sc=False, needs_layout_passes=False),
      scratch_types=[
          pltpu.VMEM_SHARED((3, B + PAD, DH), jnp.float32),
          pltpu.VMEM((B + PAD, DH), jnp.float32),
          pltpu.VMEM((CH, DH), jnp.float32),
          pltpu.VMEM((CH,), jnp.int32),
          pltpu.VMEM((1, CH), jnp.int32),
          pltpu.VMEM((80, DH), jnp.float32),
          pltpu.VMEM((80,), jnp.int32),
          pltpu.VMEM((1, 80), jnp.int32),
      ],
  )
  return kfn(h1, h2, h3, bat)


_BR = 5000


def _z_body(din, h_ref, w1_ref, b1_ref, z_ref, s_ref):
  i = pl.program_id(0)

  @pl.when(i == 0)
  def _():
    s_ref[...] = jnp.zeros_like(s_ref)

  z = jnp.dot(h_ref[...], w1_ref[...],
              preferred_element_type=jnp.float32) + b1_ref[...]
  z_ref[...] = z
  s_ref[...] += jnp.concatenate(
      [jnp.sum(z, axis=0, keepdims=True), jnp.zeros((7, DH), jnp.float32)],
      axis=0)


def _zpass(h, W1, b1, din):
  full = lambda r, c: pl.BlockSpec((r, c), lambda i: (0, 0))
  return pl.pallas_call(
      functools.partial(_z_body, din),
      grid=(N // _BR,),
      in_specs=[
          pl.BlockSpec((_BR, din), lambda i: (i, 0)),
          full(din, DH),
          full(1, DH),
      ],
      out_specs=[pl.BlockSpec((_BR, DH), lambda i: (i, 0)), full(8, DH)],
      out_shape=[jax.ShapeDtypeStruct((N, DH), jnp.float32),
                 jax.ShapeDtypeStruct((8, DH), jnp.float32)],
  )(h, W1, b1.reshape(1, DH))


def _var_body(z_ref, s_ref, v_ref):
  i = pl.program_id(0)

  @pl.when(i == 0)
  def _():
    v_ref[...] = jnp.zeros_like(v_ref)

  m = s_ref[0:1, :] * (1.0 / N)
  zc = z_ref[...] - m
  v_ref[...] += jnp.concatenate(
      [jnp.sum(zc * zc, axis=0, keepdims=True),
       jnp.zeros((7, DH), jnp.float32)], axis=0)


def _varpass(z, s):
  full = lambda r, c: pl.BlockSpec((r, c), lambda i: (0, 0))
  return pl.pallas_call(
      _var_body,
      grid=(N // _BR,),
      in_specs=[pl.BlockSpec((_BR, DH), lambda i: (i, 0)), full(8, DH)],
      out_specs=full(8, DH),
      out_shape=jax.ShapeDtypeStruct((8, DH), jnp.float32),
  )(z, s)


def _bnmlp_body(z_ref, s_ref, v_ref, g_ref, bt_ref, w2_ref, b2_ref, o_ref):
  m = s_ref[0:1, :] * (1.0 / N)
  v = v_ref[0:1, :] * (1.0 / N)
  t = jnp.maximum(
      g_ref[...] * (z_ref[...] - m) / jnp.sqrt(v + 1e-5) + bt_ref[...], 0.0)
  o_ref[...] = jnp.maximum(
      jnp.dot(t, w2_ref[...], preferred_element_type=jnp.float32) + b2_ref[...],
      0.0)


def _bnmlp(z, s, v, g, bt, W2, b2):
  full = lambda r, c: pl.BlockSpec((r, c), lambda i: (0, 0))
  return pl.pallas_call(
      _bnmlp_body,
      grid=(N // _BR,),
      in_specs=[
          pl.BlockSpec((_BR, DH), lambda i: (i, 0)),
          full(8, DH), full(8, DH), full(1, DH), full(1, DH),
          full(DH, DH), full(1, DH),
      ],
      out_specs=pl.BlockSpec((_BR, DH), lambda i: (i, 0)),
      out_shape=jax.ShapeDtypeStruct((N, DH), jnp.float32),
  )(z, s, v, g.reshape(1, DH), bt.reshape(1, DH), W2, b2.reshape(1, DH))


def _mlp(h, W1, b1, g, bt, W2, b2, din):
  z, szsum = _zpass(h, W1, b1, din)
  vsum = _varpass(z, szsum)
  return _bnmlp(z, szsum, vsum, g, bt, W2, b2)


def _heads_body(p_ref, fw1, fb1, fw2, fb2, bw1, bb1, bw2, bb2, of_ref, ob_ref):
  p = p_ref[...]
  ps = p[0] + p[1]
  h = jnp.concatenate([ps[0], ps[1], ps[2]], axis=1)
  tf = jnp.maximum(jnp.dot(h, fw1[...], preferred_element_type=jnp.float32)
                   + fb1[...], 0.0)
  of_ref[...] = jnp.dot(tf, fw2[...], preferred_element_type=jnp.float32) + fb2[...]
  tb = jnp.maximum(jnp.dot(h, bw1[...], preferred_element_type=jnp.float32)
                   + bb1[...], 0.0)
  ob_ref[...] = jnp.dot(tb, bw2[...], preferred_element_type=jnp.float32) + bb2[...]


def _heads(pooled, f_W1, f_b1, f_W2, f_b2, b_W1, b_b1, b_W2, b_b2):
  def full(*sh):
    return pl.BlockSpec(sh, lambda: tuple(0 for _ in sh))
  return pl.pallas_call(
      _heads_body,
      in_specs=[
          full(2, 3, B, DH),
          full(3 * DH, DH), full(1, DH), full(DH, NT), full(1, NT),
          full(3 * DH, DH), full(1, DH), full(DH, NT), full(1, NT),
      ],
      out_specs=[full(B, NT), full(B, NT)],
      out_shape=[jax.ShapeDtypeStruct((B, NT), jnp.float32),
                 jax.ShapeDtypeStruct((B, NT), jnp.float32)],
  )(pooled, f_W1, f_b1.reshape(1, DH), f_W2, f_b2.reshape(1, NT),
    b_W1, b_b1.reshape(1, DH), b_W2, b_b2.reshape(1, NT))


def kernel(x, edge_index, edge_attr, batch, emb,
           c1_We, c1_be, c1_W1, c1_b1, c1_g, c1_bt, c1_W2, c1_b2,
           c2_We, c2_be, c2_W1, c2_b1, c2_g, c2_bt, c2_W2, c2_b2,
           c3_We, c3_be, c3_W1, c3_b1, c3_g, c3_bt, c3_W2, c3_b2,
           f_W1, f_b1, f_W2, f_b2, b_W1, b_b1, b_W2, b_b2):
  xi = x[:, 0].astype(jnp.int32)
  src = edge_index[0].astype(jnp.int32)
  dst = edge_index[1].astype(jnp.int32)
  ea = edge_attr[:, 0].astype(jnp.float32)

  h0 = _emb_lookup(emb, xi)
  hs1 = _gine_msg(h0, src, dst, ea, c1_We[0], c1_be, DE)
  h1 = _mlp(hs1, c1_W1, c1_b1, c1_g, c1_bt, c1_W2, c1_b2, DE)
  hs2 = _gine_msg(h1, src, dst, ea, c2_We[0], c2_be, DH)
  h2 = _mlp(hs2, c2_W1, c2_b1, c2_g, c2_bt, c2_W2, c2_b2, DH)
  hs3 = _gine_msg(h2, src, dst, ea, c3_We[0], c3_be, DH)
  h3 = _mlp(hs3, c3_W1, c3_b1, c3_g, c3_bt, c3_W2, c3_b2, DH)
  pooled = _pool(h1, h2, h3, batch.astype(jnp.int32))
  return _heads(pooled, f_W1, f_b1, f_W2, f_b2, b_W1, b_b1, b_W2, b_b2)
